# baseline (speedup 1.0000x reference)
import functools

import jax
import jax.numpy as jnp
from jax import lax
from jax.experimental import pallas as pl
from jax.experimental.pallas import tpu as pltpu
from jax.experimental.pallas import tpu_sc as plsc

N_MOL = 16384
N_MOTIF = 512
D = 256
B = 8
K = 8
TN = 8192

_SC_INFO = plsc.get_sparse_core_info()
_NC = _SC_INFO.num_cores
_L = _SC_INFO.num_lanes
_N_WORKERS = B * K // _L


_NT = B - 1
_NROW = _NT * N_MOTIF // 128


def _sc_motif_gather_body(mbi_hbm, motif_hbm, out_hbm, idx_v, sidx_v, ones_v,
                          hist_sh, histv, shift_v, sel_v, rows_v, sem):
    wid = lax.axis_index("s") * _NC + lax.axis_index("c")

    @pl.when(wid == 0)
    def _work():
        pltpu.sync_copy(mbi_hbm, idx_v)
        lane = lax.iota(jnp.int32, _L)
        zero = jnp.zeros((_L,), jnp.int32)
        one = zero + 1
        histv[...] = zero
        pltpu.sync_copy(histv, hist_sh)
        for c in range(_NT * N_MOTIF // _L):
            ones_v[pl.ds(c * _L, _L)] = one
        for i in range(N_MOTIF // _L):
            chunk = idx_v[pl.ds(i * _L, _L)]
            for t in range(1, B):
                flat = (t - 1) * N_MOTIF + i * _L
                sidx_v[pl.ds(flat, _L)] = chunk + t
        pltpu.async_copy(ones_v, hist_sh.at[sidx_v], sem, add=True).wait()
        pltpu.sync_copy(hist_sh, histv)
        v1 = histv[...] * jnp.clip(B - lane, 0, 1)
        shift_v[pl.ds(0, _L)] = zero
        shift_v[pl.ds(B, _L)] = v1
        offs = v1 + shift_v[pl.ds(0, _L)]
        for g in range(B * K // _L):
            k_vec = 2 * g + lax.shift_right_logical(lane, 3)
            sel_v[pl.ds(g * _L, _L)] = jnp.minimum(offs + k_vec, N_MOTIF - 1)
        pltpu.async_copy(motif_hbm.at[sel_v], rows_v, sem).wait()
        pltpu.sync_copy(rows_v, out_hbm)


def _sc_motif_gather(motif_batch_indices, motif_atom_hiddens):
    mesh = plsc.VectorSubcoreMesh(core_axis_name="c", subcore_axis_name="s")
    run = functools.partial(
        pl.kernel,
        mesh=mesh,
        out_type=jax.ShapeDtypeStruct((B * K, D), jnp.float32),
        scratch_types=[
            pltpu.VMEM((N_MOTIF,), jnp.int32),
            pltpu.VMEM((_NT * N_MOTIF,), jnp.int32),
            pltpu.VMEM((_NT * N_MOTIF,), jnp.int32),
            pltpu.VMEM_SHARED((_L,), jnp.int32),
            pltpu.VMEM((_L,), jnp.int32),
            pltpu.VMEM((_L + B,), jnp.int32),
            pltpu.VMEM((B * K,), jnp.int32),
            pltpu.VMEM((B * K, D), jnp.float32),
            pltpu.SemaphoreType.DMA,
        ],
    )(_sc_motif_gather_body)
    return run(motif_batch_indices, motif_atom_hiddens)


def _tc_body(x_ref, seg_ref, msel_ref, w1_ref, wc_ref, b_ref,
             attach_ref, out_ref, ta_ref, tm_ref):
    pid = pl.program_id(0)

    @pl.when(pid == 0)
    def _build_tables():
        blocks = []
        for k in range(K):
            blocks.append(jax.lax.dot(msel_ref[B * k:B * (k + 1), :],
                                      w1_ref[...],
                                      preferred_element_type=jnp.float32))
        a = jnp.concatenate(blocks, axis=1)
        b_rep = jnp.concatenate([b_ref[...]] * K, axis=1)
        ta_ref[...] = a + b_rep
        r8 = jax.lax.broadcasted_iota(jnp.int32, (K, K * 4), 0)
        c32 = jax.lax.broadcasted_iota(jnp.int32, (K, K * 4), 1) // 4
        expand = (r8 == c32).astype(jnp.float32)
        tm_ref[...] = jax.lax.dot(attach_ref[...], expand,
                                  preferred_element_type=jnp.float32)

    seg = seg_ref[...]
    lanes = jax.lax.broadcasted_iota(jnp.int32, (TN, B), 1)
    oh = (seg == lanes).astype(jnp.float32)
    arows = jax.lax.dot(oh, ta_ref[...], preferred_element_type=jnp.float32)
    mrows = jax.lax.dot(oh, tm_ref[...], preferred_element_type=jnp.float32)
    acc = jax.lax.dot(x_ref[...], wc_ref[...],
                      preferred_element_type=jnp.float32)
    out_ref[...] = jnp.maximum(acc + arows, 0.0) * mrows


@jax.jit
def kernel(mol_atom_hiddens, mol_batch_indices, motif_atom_hiddens,
           motif_batch_indices, selected_attachments, W, b):
    n = mol_atom_hiddens.shape[0]
    grid = n // TN
    w1 = W[:D, :]
    w2 = W[D:, :]
    wc = jnp.tile(w2, (1, K))
    seg_col = mol_batch_indices.reshape(n, 1)
    attach_f = selected_attachments.astype(jnp.float32)
    b_row = b.reshape(1, 4)

    motif_sel = _sc_motif_gather(motif_batch_indices, motif_atom_hiddens)

    out32 = pl.pallas_call(
        _tc_body,
        grid=(grid,),
        in_specs=[
            pl.BlockSpec((TN, D), lambda i: (i, 0)),
            pl.BlockSpec((TN, 1), lambda i: (i, 0)),
            pl.BlockSpec((B * K, D), lambda i: (0, 0)),
            pl.BlockSpec((D, 4), lambda i: (0, 0)),
            pl.BlockSpec((D, K * 4), lambda i: (0, 0)),
            pl.BlockSpec((1, 4), lambda i: (0, 0)),
            pl.BlockSpec((B, K), lambda i: (0, 0)),
        ],
        out_specs=pl.BlockSpec((TN, K * 4), lambda i: (i, 0)),
        out_shape=jax.ShapeDtypeStruct((n, K * 4), jnp.float32),
        scratch_shapes=[
            pltpu.VMEM((B, K * 4), jnp.float32),
            pltpu.VMEM((B, K * 4), jnp.float32),
        ],
        compiler_params=pltpu.CompilerParams(
            dimension_semantics=("arbitrary",),
        ),
    )(mol_atom_hiddens, seg_col, motif_sel, w1, wc, b_row, attach_f)

    return out32.reshape(n, K, 4)


# speedup vs baseline: 1.0069x; 1.0069x over previous
import functools

import jax
import jax.numpy as jnp
from jax import lax
from jax.experimental import pallas as pl
from jax.experimental.pallas import tpu as pltpu
from jax.experimental.pallas import tpu_sc as plsc

N_MOL = 16384
N_MOTIF = 512
D = 256
B = 8
K = 8
TN = 8192

_SC_INFO = plsc.get_sparse_core_info()
_NC = _SC_INFO.num_cores
_L = _SC_INFO.num_lanes
_N_WORKERS = B * K // _L


def _sc_motif_gather_body(mbi_hbm, motif_hbm, out_hbm, idx_v, ones_v,
                          hist_sh, pref_sh, histv, prefv, shift_v, sel_v,
                          rows_v, sem):
    wid = lax.axis_index("s") * _NC + lax.axis_index("c")

    @pl.when(wid == 0)
    def _work():
        pltpu.sync_copy(mbi_hbm, idx_v)
        lane = lax.iota(jnp.int32, _L)
        zero = jnp.zeros((_L,), jnp.int32)
        one = zero + 1
        histv[...] = zero
        pltpu.sync_copy(histv, hist_sh)
        pltpu.sync_copy(histv, pref_sh)
        for c in range(N_MOTIF // _L):
            ones_v[pl.ds(c * _L, _L)] = one
        pltpu.async_copy(ones_v, hist_sh.at[idx_v], sem, add=True).wait()
        pltpu.sync_copy(hist_sh, histv)
        descs = []
        for t in range(1, B):
            tgt = jnp.minimum(lane + t, _L - 1)
            descs.append(
                pltpu.async_copy(histv, pref_sh.at[tgt], sem, add=True))
        for d in descs:
            d.wait()
        pltpu.sync_copy(pref_sh, prefv)
        v1 = prefv[...] * jnp.clip(B - lane, 0, 1)
        shift_v[pl.ds(0, _L)] = zero
        shift_v[pl.ds(B, _L)] = v1
        offs = v1 + shift_v[pl.ds(0, _L)]
        for g in range(B * K // _L):
            k_vec = 2 * g + lax.shift_right_logical(lane, 3)
            sel_v[pl.ds(g * _L, _L)] = jnp.minimum(offs + k_vec, N_MOTIF - 1)
        pltpu.async_copy(motif_hbm.at[sel_v], rows_v, sem).wait()
        pltpu.sync_copy(rows_v, out_hbm)


def _sc_motif_gather(motif_batch_indices, motif_atom_hiddens):
    mesh = plsc.VectorSubcoreMesh(core_axis_name="c", subcore_axis_name="s")
    run = functools.partial(
        pl.kernel,
        mesh=mesh,
        out_type=jax.ShapeDtypeStruct((B * K, D), jnp.float32),
        scratch_types=[
            pltpu.VMEM((N_MOTIF,), jnp.int32),
            pltpu.VMEM((N_MOTIF,), jnp.int32),
            pltpu.VMEM_SHARED((_L,), jnp.int32),
            pltpu.VMEM_SHARED((_L,), jnp.int32),
            pltpu.VMEM((_L,), jnp.int32),
            pltpu.VMEM((_L,), jnp.int32),
            pltpu.VMEM((_L + B,), jnp.int32),
            pltpu.VMEM((B * K,), jnp.int32),
            pltpu.VMEM((B * K, D), jnp.float32),
            pltpu.SemaphoreType.DMA,
        ],
    )(_sc_motif_gather_body)
    return run(motif_batch_indices, motif_atom_hiddens)


def _tc_body(x_ref, seg_ref, msel_ref, w1_ref, wc_ref, b_ref,
             attach_ref, out_ref, ta_ref, tm_ref):
    pid = pl.program_id(0)

    @pl.when(pid == 0)
    def _build_tables():
        blocks = []
        for k in range(K):
            blocks.append(jax.lax.dot(msel_ref[B * k:B * (k + 1), :],
                                      w1_ref[...],
                                      preferred_element_type=jnp.float32))
        a = jnp.concatenate(blocks, axis=1)
        b_rep = jnp.concatenate([b_ref[...]] * K, axis=1)
        ta_ref[...] = a + b_rep
        r8 = jax.lax.broadcasted_iota(jnp.int32, (K, K * 4), 0)
        c32 = jax.lax.broadcasted_iota(jnp.int32, (K, K * 4), 1) // 4
        expand = (r8 == c32).astype(jnp.float32)
        tm_ref[...] = jax.lax.dot(attach_ref[...], expand,
                                  preferred_element_type=jnp.float32)

    seg = seg_ref[...]
    lanes = jax.lax.broadcasted_iota(jnp.int32, (TN, B), 1)
    oh = (seg == lanes).astype(jnp.float32)
    arows = jax.lax.dot(oh, ta_ref[...], preferred_element_type=jnp.float32)
    mrows = jax.lax.dot(oh, tm_ref[...], preferred_element_type=jnp.float32)
    acc = jax.lax.dot(x_ref[...], wc_ref[...],
                      preferred_element_type=jnp.float32)
    out_ref[...] = jnp.maximum(acc + arows, 0.0) * mrows


@jax.jit
def kernel(mol_atom_hiddens, mol_batch_indices, motif_atom_hiddens,
           motif_batch_indices, selected_attachments, W, b):
    n = mol_atom_hiddens.shape[0]
    grid = n // TN
    w1 = W[:D, :]
    w2 = W[D:, :]
    wc = jnp.tile(w2, (1, K))
    seg_col = mol_batch_indices.reshape(n, 1)
    attach_f = selected_attachments.astype(jnp.float32)
    b_row = b.reshape(1, 4)

    motif_sel = _sc_motif_gather(motif_batch_indices, motif_atom_hiddens)

    out32 = pl.pallas_call(
        _tc_body,
        grid=(grid,),
        in_specs=[
            pl.BlockSpec((TN, D), lambda i: (i, 0)),
            pl.BlockSpec((TN, 1), lambda i: (i, 0)),
            pl.BlockSpec((B * K, D), lambda i: (0, 0)),
            pl.BlockSpec((D, 4), lambda i: (0, 0)),
            pl.BlockSpec((D, K * 4), lambda i: (0, 0)),
            pl.BlockSpec((1, 4), lambda i: (0, 0)),
            pl.BlockSpec((B, K), lambda i: (0, 0)),
        ],
        out_specs=pl.BlockSpec((TN, K * 4), lambda i: (i, 0)),
        out_shape=jax.ShapeDtypeStruct((n, K * 4), jnp.float32),
        scratch_shapes=[
            pltpu.VMEM((B, K * 4), jnp.float32),
            pltpu.VMEM((B, K * 4), jnp.float32),
        ],
        compiler_params=pltpu.CompilerParams(
            dimension_semantics=("arbitrary",),
        ),
    )(mol_atom_hiddens, seg_col, motif_sel, w1, wc, b_row, attach_f)

    return out32.reshape(n, K, 4)


# speedup vs baseline: 1.0086x; 1.0018x over previous
import functools

import jax
import jax.numpy as jnp
from jax import lax
from jax.experimental import pallas as pl
from jax.experimental.pallas import tpu as pltpu
from jax.experimental.pallas import tpu_sc as plsc

N_MOL = 16384
N_MOTIF = 512
D = 256
B = 8
K = 8
TN = 8192

_SC_INFO = plsc.get_sparse_core_info()
_NC = _SC_INFO.num_cores
_L = _SC_INFO.num_lanes


def _sc_motif_gather_body(mbi_hbm, motif_hbm, out_hbm, idx_v, ones_v,
                          hist_sh, pref_sh, histv, prefv, shift_v, sel_v,
                          rows_v, sem):
    wid = lax.axis_index("s") * _NC + lax.axis_index("c")

    @pl.when(wid == 0)
    def _work():
        pltpu.sync_copy(mbi_hbm, idx_v)
        lane = lax.iota(jnp.int32, _L)
        zero = jnp.zeros((_L,), jnp.int32)
        one = zero + 1
        histv[...] = zero
        pltpu.sync_copy(histv, hist_sh)
        pltpu.sync_copy(histv, pref_sh)
        for c in range(N_MOTIF // _L):
            ones_v[pl.ds(c * _L, _L)] = one
        pltpu.async_copy(ones_v, hist_sh.at[idx_v], sem, add=True).wait()
        pltpu.sync_copy(hist_sh, histv)
        descs = []
        for t in range(1, B):
            tgt = jnp.minimum(lane + t, _L - 1)
            descs.append(
                pltpu.async_copy(histv, pref_sh.at[tgt], sem, add=True))
        for d in descs:
            d.wait()
        pltpu.sync_copy(pref_sh, prefv)
        v1 = prefv[...] * jnp.clip(B - lane, 0, 1)
        shift_v[pl.ds(0, _L)] = zero
        shift_v[pl.ds(B, _L)] = v1
        offs = v1 + shift_v[pl.ds(0, _L)]
        for g in range(B * K // _L):
            k_vec = 2 * g + lax.shift_right_logical(lane, 3)
            sel_v[pl.ds(g * _L, _L)] = jnp.minimum(offs + k_vec, N_MOTIF - 1)
        pltpu.async_copy(motif_hbm.at[sel_v], rows_v, sem).wait()
        pltpu.sync_copy(rows_v, out_hbm)


def _sc_motif_gather(motif_batch_indices, motif_atom_hiddens):
    mesh = plsc.VectorSubcoreMesh(core_axis_name="c", subcore_axis_name="s")
    run = functools.partial(
        pl.kernel,
        mesh=mesh,
        out_type=jax.ShapeDtypeStruct((B * K, D), jnp.float32),
        scratch_types=[
            pltpu.VMEM((N_MOTIF,), jnp.int32),
            pltpu.VMEM((N_MOTIF,), jnp.int32),
            pltpu.VMEM_SHARED((_L,), jnp.int32),
            pltpu.VMEM_SHARED((_L,), jnp.int32),
            pltpu.VMEM((_L,), jnp.int32),
            pltpu.VMEM((_L,), jnp.int32),
            pltpu.VMEM((_L + B,), jnp.int32),
            pltpu.VMEM((B * K,), jnp.int32),
            pltpu.VMEM((B * K, D), jnp.float32),
            pltpu.SemaphoreType.DMA,
        ],
    )(_sc_motif_gather_body)
    return run(motif_batch_indices, motif_atom_hiddens)


def _tc_body(x_ref, seg_ref, msel_ref, w1_ref, wc_ref, b_ref,
             attach_ref, out_ref, ta_ref, tm_ref):
    pid = pl.program_id(0)

    @pl.when(pid == 0)
    def _build_tables():
        blocks = []
        for k in range(K):
            blocks.append(jax.lax.dot(msel_ref[B * k:B * (k + 1), :],
                                      w1_ref[...],
                                      preferred_element_type=jnp.float32))
        a = jnp.concatenate(blocks, axis=1)
        b_rep = jnp.concatenate([b_ref[...]] * K, axis=1)
        ta_ref[...] = a + b_rep
        r8 = jax.lax.broadcasted_iota(jnp.int32, (K, K * 4), 0)
        c32 = jax.lax.broadcasted_iota(jnp.int32, (K, K * 4), 1) // 4
        expand = (r8 == c32).astype(jnp.float32)
        tm_ref[...] = jax.lax.dot(attach_ref[...], expand,
                                  preferred_element_type=jnp.float32)

    seg = seg_ref[...]
    lanes = jax.lax.broadcasted_iota(jnp.int32, (TN, B), 1)
    oh = (seg == lanes).astype(jnp.float32)
    arows = jax.lax.dot(oh, ta_ref[...], preferred_element_type=jnp.float32)
    mrows = jax.lax.dot(oh, tm_ref[...], preferred_element_type=jnp.float32)
    acc = jax.lax.dot(x_ref[...], wc_ref[...],
                      preferred_element_type=jnp.float32)
    out_ref[...] = jnp.maximum(acc + arows, 0.0) * mrows


@jax.jit
def kernel(mol_atom_hiddens, mol_batch_indices, motif_atom_hiddens,
           motif_batch_indices, selected_attachments, W, b):
    n = mol_atom_hiddens.shape[0]
    grid = n // TN
    w1 = W[:D, :]
    w2 = W[D:, :]
    wc = jnp.tile(w2, (1, K))
    seg_col = mol_batch_indices.reshape(n, 1)
    attach_f = selected_attachments.astype(jnp.float32)
    b_row = b.reshape(1, 4)

    motif_sel = _sc_motif_gather(motif_batch_indices, motif_atom_hiddens)

    out32 = pl.pallas_call(
        _tc_body,
        grid=(grid,),
        in_specs=[
            pl.BlockSpec((TN, D), lambda i: (i, 0)),
            pl.BlockSpec((TN, 1), lambda i: (i, 0)),
            pl.BlockSpec((B * K, D), lambda i: (0, 0)),
            pl.BlockSpec((D, 4), lambda i: (0, 0)),
            pl.BlockSpec((D, K * 4), lambda i: (0, 0)),
            pl.BlockSpec((1, 4), lambda i: (0, 0)),
            pl.BlockSpec((B, K), lambda i: (0, 0)),
        ],
        out_specs=pl.BlockSpec((TN, K * 4), lambda i: (i, 0)),
        out_shape=jax.ShapeDtypeStruct((n, K * 4), jnp.float32),
        scratch_shapes=[
            pltpu.VMEM((B, K * 4), jnp.float32),
            pltpu.VMEM((B, K * 4), jnp.float32),
        ],
        compiler_params=pltpu.CompilerParams(
            dimension_semantics=("arbitrary",),
        ),
    )(mol_atom_hiddens, seg_col, motif_sel, w1, wc, b_row, attach_f)

    return out32.reshape(n, K, 4)
